# Initial kernel scaffold; baseline (speedup 1.0000x reference)
import jax
import jax.numpy as jnp
from jax.experimental import pallas as pl


def kernel(x, edge_index, W1, att_src1, att_dst1, b1, W2, att_src2, att_dst2, b2, Wr, br):
    raise NotImplementedError("write your pallas kernel here")


# speedup vs baseline: 10.5567x; 10.5567x over previous
import functools

import jax
import jax.numpy as jnp
from jax import lax
from jax.experimental import pallas as pl
from jax.experimental.pallas import tpu as pltpu
from jax.experimental.pallas import tpu_sc as plsc

N = 10000
E = 160000
F = 1024
K1 = 256
NCH = 8
CW = 128
NC = 2
NS = 16
B = 128

BN = 1000

_ROWS_T = 624
_TAIL = N - NS * _ROWS_T
_TBASE = NS * _ROWS_T


@functools.lru_cache(maxsize=None)
def _mesh():
    return plsc.VectorSubcoreMesh(core_axis_name="c", subcore_axis_name="s")


def _mm_body(x_ref, w_ref, as_ref, ad_ref, hc_ref, acs_ref, acd_ref):
    x = x_ref[...]
    acc_s = jnp.zeros((BN, 16), jnp.float32)
    acc_d = jnp.zeros((BN, 16), jnp.float32)
    for c in range(NCH):
        hcol = jnp.dot(x, w_ref[:, c * CW:(c + 1) * CW],
                       preferred_element_type=jnp.float32)
        hc_ref[c] = hcol
        acc_s = acc_s + jnp.dot(hcol, as_ref[c * CW:(c + 1) * CW, :],
                                preferred_element_type=jnp.float32)
        acc_d = acc_d + jnp.dot(hcol, ad_ref[c * CW:(c + 1) * CW, :],
                                preferred_element_type=jnp.float32)
    acs_ref[...] = acc_s
    acd_ref[...] = acc_d


def _tc_matmul(x, W, As, Ad, K):
    return pl.pallas_call(
        _mm_body,
        grid=(N // BN,),
        in_specs=[
            pl.BlockSpec((BN, K), lambda i: (i, 0)),
            pl.BlockSpec((K, F), lambda i: (0, 0)),
            pl.BlockSpec((F, 16), lambda i: (0, 0)),
            pl.BlockSpec((F, 16), lambda i: (0, 0)),
        ],
        out_specs=[
            pl.BlockSpec((NCH, BN, CW), lambda i: (0, i, 0)),
            pl.BlockSpec((BN, 16), lambda i: (i, 0)),
            pl.BlockSpec((BN, 16), lambda i: (i, 0)),
        ],
        out_shape=[
            jax.ShapeDtypeStruct((NCH, N, CW), jnp.float32),
            jax.ShapeDtypeStruct((N, 16), jnp.float32),
            jax.ShapeDtypeStruct((N, 16), jnp.float32),
        ],
    )(x, W, As, Ad)


_EPT = E // (NC * NS)
_NB_A = _EPT // B
_REM_A = _EPT - _NB_A * B


@functools.lru_cache(maxsize=None)
def _sc_edge_ex():
    return pl.kernel(
        _sc_edge_ex_impl,
        mesh=_mesh(),
        compiler_params=pltpu.CompilerParams(use_tc_tiling_on_sc=False),
        out_type=[
            jax.ShapeDtypeStruct((E, 16), jnp.float32),
            jax.ShapeDtypeStruct((NC, N, 16), jnp.float32),
        ],
        scratch_types=[
            pltpu.VMEM((B,), jnp.int32),
            pltpu.VMEM((B,), jnp.int32),
            pltpu.VMEM((B, 16), jnp.float32),
            pltpu.VMEM((B, 16), jnp.float32),
            pltpu.VMEM((B, 16), jnp.float32),
            pltpu.VMEM((_REM_A,), jnp.int32),
            pltpu.VMEM((_REM_A,), jnp.int32),
            pltpu.VMEM((_REM_A, 16), jnp.float32),
            pltpu.VMEM((_REM_A, 16), jnp.float32),
            pltpu.VMEM((_REM_A, 16), jnp.float32),
            pltpu.VMEM((_ROWS_T, 16), jnp.float32),
            pltpu.VMEM_SHARED((N, 16), jnp.float32),
        ],
    )


def _sc_edge_ex_impl(src_hbm, dst_hbm, acs_hbm, acd_hbm, ex_hbm, den_hbm,
                     src_v, dst_v, gs_v, gd_v, ex_v,
                     srcr_v, dstr_v, gsr_v, gdr_v, exr_v,
                     zero_v, den_acc):
    cid = lax.axis_index("c")
    sid = lax.axis_index("s")
    tbase = (cid * NS + sid) * _EPT

    iota = lax.iota(jnp.int32, 16)
    head_mask = iota < 4
    zf = jnp.zeros((16,), jnp.float32)

    def _z16(i, _):
        zero_v[i, :] = zf
        return 0
    lax.fori_loop(0, _ROWS_T, _z16, 0)

    pltpu.sync_copy(zero_v, den_acc.at[pl.ds(sid * _ROWS_T, _ROWS_T)])

    @pl.when(sid == 0)
    def _():
        pltpu.sync_copy(zero_v.at[pl.ds(0, _TAIL)],
                        den_acc.at[pl.ds(_TBASE, _TAIL)])
    plsc.subcore_barrier()

    def _edges(n_e, sref, dref, gsref, gdref, eref, eb):
        pltpu.sync_copy(src_hbm.at[pl.ds(eb, n_e)], sref)
        pltpu.sync_copy(dst_hbm.at[pl.ds(eb, n_e)], dref)
        pltpu.sync_copy(acs_hbm.at[sref], gsref)
        pltpu.sync_copy(acd_hbm.at[dref], gdref)

        def _one(i, _):
            alpha = gsref[i, :] + gdref[i, :]
            ex = jnp.exp(jnp.maximum(alpha, alpha * jnp.float32(0.2)))
            eref[i, :] = jnp.where(head_mask, ex, zf)
            return 0
        lax.fori_loop(0, n_e, _one, 0)
        pltpu.sync_copy(eref, ex_hbm.at[pl.ds(eb, n_e)])
        pltpu.sync_copy(eref, den_acc.at[dref], add=True)

    def _batch(b, _):
        _edges(B, src_v, dst_v, gs_v, gd_v, ex_v, tbase + b * B)
        return 0
    lax.fori_loop(0, _NB_A, _batch, 0)
    _edges(_REM_A, srcr_v, dstr_v, gsr_v, gdr_v, exr_v, tbase + _NB_A * B)

    plsc.subcore_barrier()
    pltpu.sync_copy(den_acc.at[pl.ds(sid * _ROWS_T, _ROWS_T)],
                    den_hbm.at[cid].at[pl.ds(sid * _ROWS_T, _ROWS_T)])

    @pl.when(sid == 0)
    def _():
        pltpu.sync_copy(den_acc.at[pl.ds(_TBASE, _TAIL)],
                        den_hbm.at[cid].at[pl.ds(_TBASE, _TAIL)])


_EPT_C = E // NS
_NB_C = _EPT_C // B
_REM_C = _EPT_C - _NB_C * B
_ZR = 208


@functools.lru_cache(maxsize=None)
def _sc_aggregate():
    return pl.kernel(
        _sc_aggregate_impl,
        mesh=_mesh(),
        compiler_params=pltpu.CompilerParams(use_tc_tiling_on_sc=False),
        out_type=jax.ShapeDtypeStruct((NCH * N, CW), jnp.float32),
        scratch_types=[
            pltpu.VMEM((B,), jnp.int32),
            pltpu.VMEM((B,), jnp.int32),
            pltpu.VMEM((B,), jnp.int32),
            pltpu.VMEM((B, 16), jnp.float32),
            pltpu.VMEM((B, CW), jnp.float32),
            pltpu.VMEM((_REM_C,), jnp.int32),
            pltpu.VMEM((_REM_C,), jnp.int32),
            pltpu.VMEM((_REM_C,), jnp.int32),
            pltpu.VMEM((_REM_C, 16), jnp.float32),
            pltpu.VMEM((_REM_C, CW), jnp.float32),
            pltpu.VMEM((_ZR, CW), jnp.float32),
            pltpu.VMEM_SHARED((N, CW), jnp.float32),
            pltpu.SemaphoreType.DMA,
        ],
    )


def _sc_aggregate_impl(hflat_hbm, src_hbm, dst_hbm, ex_hbm, out_hbm,
                       src_v, dst_v, idx_v, ex_v, g_v,
                       srcr_v, dstr_v, idxr_v, exr_v, gr_v,
                       zero_v, acc, sem):
    cid = lax.axis_index("c")
    sid = lax.axis_index("s")
    tbase = sid * _EPT_C
    iota = lax.iota(jnp.int32, 16)
    zf = jnp.zeros((16,), jnp.float32)

    def _zrow(i, _):
        for jj in range(CW // 16):
            zero_v[i, pl.ds(jj * 16, 16)] = zf
        return 0
    lax.fori_loop(0, _ZR, _zrow, 0)

    def _mk_batch(n_e, sref, dref, iref, eref, gref):
        def _run(chunk, hd0, eb):
            pltpu.sync_copy(src_hbm.at[pl.ds(eb, n_e)], sref)
            pltpu.sync_copy(dst_hbm.at[pl.ds(eb, n_e)], dref)
            pltpu.sync_copy(ex_hbm.at[pl.ds(eb, n_e)], eref)
            off = chunk * N
            for jj in range(n_e // 16):
                sl = pl.ds(jj * 16, 16)
                iref[sl] = sref[sl] + off
            pltpu.async_copy(hflat_hbm.at[iref], gref, sem).wait()

            def _scale(i, _):
                v = eref[i, :]
                s = jnp.where(cid == 0, v[hd0], v[hd0 + 2])
                for jj in range(CW // 16):
                    sl = pl.ds(jj * 16, 16)
                    gref[i, sl] = gref[i, sl] * s
                return 0
            lax.fori_loop(0, n_e, _scale, 0)
            pltpu.sync_copy(gref, acc.at[dref], add=True)
        return _run

    _full = _mk_batch(B, src_v, dst_v, idx_v, ex_v, g_v)
    _rem = _mk_batch(_REM_C, srcr_v, dstr_v, idxr_v, exr_v, gr_v)

    for cc in range(NCH // NC):
        chunk = cid * (NCH // NC) + cc
        hd0 = cc // 2

        for k in range(_ROWS_T // _ZR):
            pltpu.sync_copy(
                zero_v, acc.at[pl.ds(sid * _ROWS_T + k * _ZR, _ZR)])

        @pl.when(sid == 0)
        def _():
            pltpu.sync_copy(zero_v.at[pl.ds(0, _TAIL)],
                            acc.at[pl.ds(_TBASE, _TAIL)])
        plsc.subcore_barrier()

        def _batch(b, _):
            _full(chunk, hd0, tbase + b * B)
            return 0
        lax.fori_loop(0, _NB_C, _batch, 0)
        _rem(chunk, hd0, tbase + _NB_C * B)

        plsc.subcore_barrier()
        pltpu.sync_copy(
            acc.at[pl.ds(sid * _ROWS_T, _ROWS_T)],
            out_hbm.at[pl.ds(chunk * N + sid * _ROWS_T, _ROWS_T)])

        @pl.when(sid == 0)
        def _():
            pltpu.sync_copy(acc.at[pl.ds(_TBASE, _TAIL)],
                            out_hbm.at[pl.ds(chunk * N + _TBASE, _TAIL)])
        plsc.subcore_barrier()


def _comb_chunks(S_ref, h_ref, acs_ref, acd_ref, dpart_ref, b_ref):
    asum = acs_ref[:, 0:4] + acd_ref[:, 0:4]
    exself = jnp.exp(jnp.maximum(asum, asum * 0.2))
    dtot = dpart_ref[0, :, 0:4] + dpart_ref[1, :, 0:4] + exself
    inv = 1.0 / dtot
    outs = []
    for c in range(NCH):
        hd = c // 2
        v = ((S_ref[c] + exself[:, hd:hd + 1] * h_ref[c]) * inv[:, hd:hd + 1]
             + b_ref[0, c * CW:(c + 1) * CW])
        outs.append(jnp.where(v > 0, v, jnp.exp(jnp.minimum(v, 0.0)) - 1.0))
    return outs


def _comb1_body(S_ref, h_ref, acs_ref, acd_ref, dpart_ref, b_ref, y_ref):
    outs = _comb_chunks(S_ref, h_ref, acs_ref, acd_ref, dpart_ref, b_ref)
    for c in range(NCH):
        y_ref[:, c * CW:(c + 1) * CW] = outs[c]


def _comb2_body(S_ref, h_ref, acs_ref, acd_ref, dpart_ref, b_ref, wr_ref,
                br_ref, out_ref):
    outs = _comb_chunks(S_ref, h_ref, acs_ref, acd_ref, dpart_ref, b_ref)
    acc = jnp.zeros((BN, 1), jnp.float32)
    for c in range(NCH):
        acc = acc + jnp.dot(outs[c], wr_ref[c * CW:(c + 1) * CW, :],
                            preferred_element_type=jnp.float32)
    out_ref[...] = acc + br_ref[0, :]


_COMB_SPECS = [
    pl.BlockSpec((NCH, BN, CW), lambda i: (0, i, 0)),
    pl.BlockSpec((NCH, BN, CW), lambda i: (0, i, 0)),
    pl.BlockSpec((BN, 16), lambda i: (i, 0)),
    pl.BlockSpec((BN, 16), lambda i: (i, 0)),
    pl.BlockSpec((NC, BN, 16), lambda i: (0, i, 0)),
    pl.BlockSpec((1, F), lambda i: (0, 0)),
]


def _tc_combine1(S, hc, acs, acd, dpart, b):
    return pl.pallas_call(
        _comb1_body,
        grid=(N // BN,),
        in_specs=_COMB_SPECS,
        out_specs=pl.BlockSpec((BN, F), lambda i: (i, 0)),
        out_shape=jax.ShapeDtypeStruct((N, F), jnp.float32),
    )(S, hc, acs, acd, dpart, b)


def _tc_combine2(S, hc, acs, acd, dpart, b, Wr, br):
    return pl.pallas_call(
        _comb2_body,
        grid=(N // BN,),
        in_specs=_COMB_SPECS + [
            pl.BlockSpec((F, 1), lambda i: (0, 0)),
            pl.BlockSpec((1, 1), lambda i: (0, 0)),
        ],
        out_specs=pl.BlockSpec((BN, 1), lambda i: (i, 0)),
        out_shape=jax.ShapeDtypeStruct((N, 1), jnp.float32),
    )(S, hc, acs, acd, dpart, b, Wr, br)


def _block_diag_att(att):
    eye = jnp.eye(4, 16, dtype=jnp.float32)
    return jnp.einsum('hc,hk->hck', att, eye).reshape(F, 16)


def _gat_layer(x, src, dst, W, att_s, att_d, K):
    As = _block_diag_att(att_s)
    Ad = _block_diag_att(att_d)
    hc, acs, acd = _tc_matmul(x, W, As, Ad, K)
    ex, dpart = _sc_edge_ex()(src, dst, acs, acd)
    S = _sc_aggregate()(hc.reshape(NCH * N, CW), src, dst, ex)
    return S.reshape(NCH, N, CW), hc, acs, acd, dpart


def kernel(x, edge_index, W1, att_src1, att_dst1, b1, W2, att_src2, att_dst2,
           b2, Wr, br):
    src = edge_index[0]
    dst = edge_index[1]

    S1, hc1, acs1, acd1, dp1 = _gat_layer(x, src, dst, W1, att_src1,
                                          att_dst1, K1)
    y1 = _tc_combine1(S1, hc1, acs1, acd1, dp1, b1.reshape(1, F))
    S2, hc2, acs2, acd2, dp2 = _gat_layer(y1, src, dst, W2, att_src2,
                                          att_dst2, F)
    return _tc_combine2(S2, hc2, acs2, acd2, dp2, b2.reshape(1, F), Wr,
                        br.reshape(1, 1))
